# Initial kernel scaffold; baseline (speedup 1.0000x reference)
import jax
import jax.numpy as jnp
from jax.experimental import pallas as pl


def kernel(x, edge_index, pos, batch, W_e1, b_e1, W_e2, b_e2, W_c1, b_c1, W_c2, b_c2, W_n1, b_n1, W_n2, b_n2, bn_gamma, bn_beta, W_fc, b_fc):
    raise NotImplementedError("write your pallas kernel here")


# speedup vs baseline: 1.9773x; 1.9773x over previous
import functools

import jax
import jax.numpy as jnp
from jax import lax
from jax.experimental import pallas as pl
from jax.experimental.pallas import tpu as pltpu
import jax.experimental.pallas.tpu_sc as plsc

N = 10000
E = 320000
F = 128
MDIM = 16
NC, NS = 2, 16
NW = NC * NS
CHUNK = 128
CH_PER_W = 80
EPAD = NW * CH_PER_W * CHUNK
NPAD = 10112
DUMMY = 10008
STRIPE = NPAD // NS
EB = 4096
NBLK = EPAD // EB


def _sc_gather(x, px, py, pz, src_idx, dst_idx):
    mesh = plsc.VectorSubcoreMesh(core_axis_name="c", subcore_axis_name="s",
                                  num_cores=NC, num_subcores=NS)

    @functools.partial(
        pl.kernel,
        out_type=[jax.ShapeDtypeStruct((EPAD, F), jnp.float32),
                  jax.ShapeDtypeStruct((EPAD, F), jnp.float32),
                  jax.ShapeDtypeStruct((EPAD // 16, 16), jnp.float32)],
        mesh=mesh,
        compiler_params=pltpu.CompilerParams(needs_layout_passes=False),
        scratch_types=[pltpu.VMEM((CHUNK,), jnp.int32),
                       pltpu.VMEM((CHUNK,), jnp.int32),
                       pltpu.VMEM((CHUNK, F), jnp.float32),
                       pltpu.VMEM((CHUNK, F), jnp.float32),
                       pltpu.VMEM((CHUNK // 16, 16), jnp.float32),
                       pltpu.VMEM((NPAD,), jnp.float32),
                       pltpu.VMEM((NPAD,), jnp.float32),
                       pltpu.VMEM((NPAD,), jnp.float32),
                       pltpu.SemaphoreType.DMA,
                       pltpu.SemaphoreType.DMA],
    )
    def k(x_hbm, px_hbm, py_hbm, pz_hbm, src_hbm, dst_hbm,
          xi_hbm, xj_hbm, rel_hbm,
          idx_a, idx_b, rows_a, rows_b, rel_v, px_v, py_v, pz_v,
          sem_a, sem_b):
        wid = lax.axis_index("c") * NS + lax.axis_index("s")
        base = wid * CH_PER_W * CHUNK
        pltpu.sync_copy(px_hbm, px_v)
        pltpu.sync_copy(py_hbm, py_v)
        pltpu.sync_copy(pz_hbm, pz_v)

        @pl.loop(0, CH_PER_W)
        def _(j):
            off = base + j * CHUNK
            pltpu.sync_copy(src_hbm.at[wid, j], idx_a)
            pltpu.sync_copy(dst_hbm.at[wid, j], idx_b)
            ca = pltpu.async_copy(x_hbm.at[idx_a], rows_a, sem_a)
            cb = pltpu.async_copy(x_hbm.at[idx_b], rows_b, sem_b)
            for q in range(CHUNK // 16):
                sv = idx_a[pl.ds(q * 16, 16)]
                dv = idx_b[pl.ds(q * 16, 16)]
                acc = jnp.zeros((16,), jnp.float32)
                for comp in (px_v, py_v, pz_v):
                    d = plsc.load_gather(comp, [sv]) - plsc.load_gather(comp, [dv])
                    acc = acc + d * d
                rel_v[q] = acc
            pltpu.sync_copy(rel_v, rel_hbm.at[pl.ds((wid * CH_PER_W + j) * 8, 8)])
            ca.wait()
            cb.wait()
            pltpu.sync_copy(rows_a, xj_hbm.at[pl.ds(off, CHUNK)])
            pltpu.sync_copy(rows_b, xi_hbm.at[pl.ds(off, CHUNK)])

    return k(x, px, py, pz, src_idx, dst_idx)


G16 = NW // 2
EG = EPAD // G16
CH2 = 2048
CPG = EG // CH2
HM = MDIM // 2
PR = 640
NPACK = PR * 16


def _sc_scatter(mt, dst_flat, zpk):
    mesh = plsc.VectorSubcoreMesh(core_axis_name="c", subcore_axis_name="s",
                                  num_cores=NC, num_subcores=NS)

    @functools.partial(
        pl.kernel,
        out_type=jax.ShapeDtypeStruct((2, G16, PR, 128), jnp.float32),
        mesh=mesh,
        compiler_params=pltpu.CompilerParams(needs_layout_passes=False),
        scratch_types=[pltpu.VMEM((PR, 128), jnp.float32),
                       pltpu.VMEM((1, CH2), jnp.int32),
                       pltpu.VMEM((HM, CH2), jnp.float32)],
    )
    def k(mt_hbm, dst_hbm, zpk_hbm, parts_hbm, acc_v, idx_v, rows_v):
        wid = lax.axis_index("c") * NS + lax.axis_index("s")
        h = wid % 2
        g = wid // 2
        pltpu.sync_copy(zpk_hbm, acc_v)

        @pl.loop(0, CPG)
        def _(kk):
            off = g * EG + kk * CH2
            pltpu.sync_copy(dst_hbm.at[pl.ds(off, CH2)], idx_v.at[0])
            pltpu.sync_copy(mt_hbm.at[pl.ds(h * HM, HM), pl.ds(off, CH2)],
                            rows_v)

            @pl.loop(0, CH2 // 16)
            def _(q):
                dvals = idx_v[0, pl.ds(q * 16, 16)]
                rr = lax.rem(dvals, PR)
                cbase = lax.div(dvals, PR) * 8
                for c in range(HM):
                    vals = rows_v[c, pl.ds(q * 16, 16)]
                    plsc.addupdate_scatter(acc_v, [rr, cbase + c], vals)

        pltpu.sync_copy(acc_v, parts_hbm.at[h, g])

    return k(mt, dst_flat, zpk)


def _edge_body(xi_ref, xj_ref, rel_ref, w1a_ref, w1b_ref, wr16_ref,
               b1_ref, w2_ref, b2_ref, m_ref):
    rb = jnp.reshape(
        jnp.broadcast_to(rel_ref[...][:, None, :], (EB // 16, 16, 16)),
        (EB, 16))
    lane = lax.broadcasted_iota(jnp.int32, (EB, 16), 1)
    row = lax.broadcasted_iota(jnp.int32, (EB, 16), 0)
    reldiag = jnp.where(lane == row % 16, rb, 0.0)
    h = (jnp.dot(xi_ref[...], w1a_ref[...], preferred_element_type=jnp.float32)
         + jnp.dot(xj_ref[...], w1b_ref[...], preferred_element_type=jnp.float32)
         + jnp.dot(reldiag, wr16_ref[...], preferred_element_type=jnp.float32)
         + b1_ref[...])
    h = h * jax.nn.sigmoid(h)
    m = jnp.dot(h, w2_ref[...], preferred_element_type=jnp.float32) + b2_ref[...]
    m = m * jax.nn.sigmoid(m)
    m_ref[...] = jnp.swapaxes(m, 0, 1)


def _node_body(x_ref, parts_ref, w1a_ref, w1b_ref, b1_ref, w2_ref, b2_ref,
               g_ref, bt_ref, wfc_ref, bfc_ref, out_ref):
    x = x_ref[...]
    p0 = jnp.sum(parts_ref[0], axis=0)
    p1 = jnp.sum(parts_ref[1], axis=0)
    mh0 = jnp.concatenate([p0[:, b * 8:(b + 1) * 8] for b in range(16)], axis=0)
    mh1 = jnp.concatenate([p1[:, b * 8:(b + 1) * 8] for b in range(16)], axis=0)
    m_i = jnp.concatenate([mh0[:N], mh1[:N]], axis=1)
    nh = (jnp.dot(x, w1a_ref[...], preferred_element_type=jnp.float32)
          + jnp.dot(m_i, w1b_ref[...], preferred_element_type=jnp.float32)
          + b1_ref[...])
    nh = nh * jax.nn.sigmoid(nh)
    hidden = jnp.dot(nh, w2_ref[...], preferred_element_type=jnp.float32) + b2_ref[...]
    h0 = x + hidden
    mean = jnp.mean(h0, axis=0, keepdims=True)
    var = jnp.mean((h0 - mean) * (h0 - mean), axis=0, keepdims=True)
    hn = (h0 - mean) * lax.rsqrt(var + 1e-5) * g_ref[...] + bt_ref[...]
    out_ref[...] = (jnp.dot(hn, wfc_ref[...], preferred_element_type=jnp.float32)
                    + bfc_ref[...])


def _edge_mlp(xi, xj, rel, w1a, w1b, wr16, b1, w2, b2):
    return pl.pallas_call(
        _edge_body,
        grid=(NBLK,),
        in_specs=[
            pl.BlockSpec((EB, F), lambda i: (i, 0)),
            pl.BlockSpec((EB, F), lambda i: (i, 0)),
            pl.BlockSpec((EB // 16, 16), lambda i: (i, 0)),
            pl.BlockSpec(w1a.shape, lambda i: (0, 0)),
            pl.BlockSpec(w1b.shape, lambda i: (0, 0)),
            pl.BlockSpec(wr16.shape, lambda i: (0, 0)),
            pl.BlockSpec(b1.shape, lambda i: (0, 0)),
            pl.BlockSpec(w2.shape, lambda i: (0, 0)),
            pl.BlockSpec(b2.shape, lambda i: (0, 0)),
        ],
        out_specs=pl.BlockSpec((MDIM, EB), lambda i: (0, i)),
        out_shape=jax.ShapeDtypeStruct((MDIM, EPAD), jnp.float32),
    )(xi, xj, rel, w1a, w1b, wr16, b1, w2, b2)


def _node_mlp(x, parts, w1a, w1b, b1, w2, b2, g, bt, wfc, bfc):
    return pl.pallas_call(
        _node_body,
        out_shape=jax.ShapeDtypeStruct((N, 3), jnp.float32),
    )(x, parts, w1a, w1b, b1, w2, b2, g, bt, wfc, bfc)


def kernel(x, edge_index, pos, batch, W_e1, b_e1, W_e2, b_e2, W_c1, b_c1,
           W_c2, b_c2, W_n1, b_n1, W_n2, b_n2, bn_gamma, bn_beta, W_fc, b_fc):
    del batch, W_c1, b_c1, W_c2, b_c2

    px = jnp.pad(pos[:, 0], (0, NPAD - N))
    py = jnp.pad(pos[:, 1], (0, NPAD - N))
    pz = jnp.pad(pos[:, 2], (0, NPAD - N))
    src = edge_index[0]
    dst = edge_index[1]
    pad = EPAD - E
    src_g = jnp.concatenate([src, jnp.zeros((pad,), jnp.int32)]
                            ).reshape(NW, CH_PER_W, CHUNK)
    dst_g = jnp.concatenate([dst, jnp.zeros((pad,), jnp.int32)]
                            ).reshape(NW, CH_PER_W, CHUNK)
    dst_s = jnp.concatenate([dst, jnp.full((pad,), DUMMY, jnp.int32)])
    zpk = jnp.zeros((PR, 128), jnp.float32)

    w1a = W_e1[:, :F].T
    w1b = W_e1[:, F:2 * F].T
    wr16 = jnp.broadcast_to(W_e1[:, 2 * F:].T, (16, 514))
    b1 = b_e1.reshape(1, -1)
    w2 = W_e2.T
    b2 = b_e2.reshape(1, -1)
    n1a = W_n1[:, :F].T
    n1b = W_n1[:, F:].T
    nb1 = b_n1.reshape(1, -1)
    n2 = W_n2.T
    nb2 = b_n2.reshape(1, -1)
    g = bn_gamma.reshape(1, -1)
    bt = bn_beta.reshape(1, -1)
    wfc = W_fc.T
    bfc = b_fc.reshape(1, -1)

    xi, xj, rel = _sc_gather(x, px, py, pz, src_g, dst_g)
    mt = _edge_mlp(xi, xj, rel, w1a, w1b, wr16, b1, w2, b2)
    parts = _sc_scatter(mt, dst_s, zpk)
    return _node_mlp(x, parts, n1a, n1b, nb1, n2, nb2, g, bt, wfc, bfc)
